# baseline (speedup 1.0000x reference)
import functools

import jax
import jax.numpy as jnp
from jax import lax
from jax.experimental import pallas as pl
from jax.experimental.pallas import tpu as pltpu
from jax.experimental.pallas import tpu_sc as plsc

D = 64
NC = 2
NS = 16
NW = NC * NS


@functools.partial(jax.jit, static_argnames=("n", "chunk"))
def _sc_gather(idx, table, n, chunk):
    b_per_w = n // NW
    nchunk = b_per_w // chunk
    mesh = plsc.VectorSubcoreMesh(core_axis_name="c", subcore_axis_name="s")

    @functools.partial(
        pl.kernel,
        out_type=jax.ShapeDtypeStruct((n, D), jnp.float32),
        mesh=mesh,
        scratch_types=[
            pltpu.VMEM((chunk,), jnp.int32),
            pltpu.VMEM((chunk, D), jnp.float32),
            pltpu.SemaphoreType.DMA,
        ],
        compiler_params=pltpu.CompilerParams(use_tc_tiling_on_sc=False),
    )
    def k(idx_hbm, table_hbm, out_hbm, idx_v, rows_v, sem):
        wid = lax.axis_index("s") * NC + lax.axis_index("c")
        base = wid * b_per_w

        def body(g, carry):
            off = base + g * chunk
            pltpu.sync_copy(idx_hbm.at[pl.ds(off, chunk)], idx_v)
            pltpu.async_copy(table_hbm.at[idx_v], rows_v, sem).wait()
            pltpu.sync_copy(rows_v, out_hbm.at[pl.ds(off, chunk)])
            return carry

        lax.fori_loop(0, nchunk, body, 0)

    return k(idx, table)


def kernel(x, table):
    n = x.shape[0] * x.shape[1]
    idx = x.reshape(-1).astype(jnp.int32)
    out = _sc_gather(idx, table, n, 512)
    return out.reshape(x.shape + (table.shape[1],))


# speedup vs baseline: 1.0461x; 1.0461x over previous
import functools

import jax
import jax.numpy as jnp
from jax import lax
from jax.experimental import pallas as pl
from jax.experimental.pallas import tpu as pltpu
from jax.experimental.pallas import tpu_sc as plsc

D = 64
NC = 2
NS = 16
NW = NC * NS
CHUNK = 256
NBUF = 4


@functools.partial(jax.jit, static_argnames=("n",))
def _sc_gather(idx, table, n):
    b_per_w = n // NW
    nchunk = b_per_w // CHUNK
    ngroup = nchunk // NBUF
    mesh = plsc.VectorSubcoreMesh(core_axis_name="c", subcore_axis_name="s")

    scratch = ([pltpu.VMEM((CHUNK,), jnp.int32) for _ in range(NBUF)]
               + [pltpu.VMEM((CHUNK, D), jnp.float32) for _ in range(NBUF)]
               + [pltpu.SemaphoreType.DMA((NBUF,)),
                  pltpu.SemaphoreType.DMA((NBUF,))])

    @functools.partial(
        pl.kernel,
        out_type=jax.ShapeDtypeStruct((n, D), jnp.float32),
        mesh=mesh,
        scratch_types=scratch,
        compiler_params=pltpu.CompilerParams(use_tc_tiling_on_sc=False),
    )
    def k(idx_hbm, table_hbm, out_hbm, *rest):
        idx_v = rest[:NBUF]
        rows_v = rest[NBUF:2 * NBUF]
        gsem, wsem = rest[2 * NBUF], rest[2 * NBUF + 1]
        wid = lax.axis_index("s") * NC + lax.axis_index("c")
        base = wid * (n // NW)

        def load_idx(g, b):
            pltpu.sync_copy(idx_hbm.at[pl.ds(base + g * CHUNK, CHUNK)],
                            idx_v[b])

        def start_gather(b):
            pltpu.async_copy(table_hbm.at[idx_v[b]], rows_v[b], gsem.at[b])

        def wait_gather(b):
            pltpu.make_async_copy(table_hbm.at[idx_v[b]], rows_v[b],
                                  gsem.at[b]).wait()

        def start_write(g, b):
            pltpu.async_copy(rows_v[b],
                             out_hbm.at[pl.ds(base + g * CHUNK, CHUNK)],
                             wsem.at[b])

        def wait_write(g, b):
            pltpu.make_async_copy(rows_v[b],
                                  out_hbm.at[pl.ds(base + g * CHUNK, CHUNK)],
                                  wsem.at[b]).wait()

        for b in range(NBUF):
            load_idx(b, b)
            start_gather(b)

        def body(p, carry):
            for b in range(NBUF):
                g = p * NBUF + b
                wait_gather(b)
                start_write(g, b)
                load_idx(g + NBUF, b)
                wait_write(g, b)
                start_gather(b)
            return carry

        lax.fori_loop(0, ngroup - 1, body, 0)

        g0 = (ngroup - 1) * NBUF
        for b in range(NBUF):
            wait_gather(b)
            start_write(g0 + b, b)
        for b in range(NBUF):
            wait_write(g0 + b, b)

    return k(idx, table)


def kernel(x, table):
    n = x.shape[0] * x.shape[1]
    idx = x.reshape(-1).astype(jnp.int32)
    out = _sc_gather(idx, table, n)
    return out.reshape(x.shape + (table.shape[1],))


# speedup vs baseline: 1.2771x; 1.2208x over previous
import functools

import jax
import jax.numpy as jnp
from jax import lax
from jax.experimental import pallas as pl
from jax.experimental.pallas import tpu as pltpu
from jax.experimental.pallas import tpu_sc as plsc

D = 64
DP = 128
NC = 2
NS = 16
NW = NC * NS
CHUNK = 128
NBUF = 4


@functools.partial(jax.jit, static_argnames=("n",))
def _sc_gather(idx, tabp, n):
    b_per_w = n // NW
    nchunk = b_per_w // CHUNK
    ngroup = nchunk // NBUF
    mesh = plsc.VectorSubcoreMesh(core_axis_name="c", subcore_axis_name="s")

    scratch = ([pltpu.VMEM((CHUNK,), jnp.int32) for _ in range(NBUF)]
               + [pltpu.VMEM((CHUNK, DP), jnp.float32) for _ in range(NBUF)]
               + [pltpu.SemaphoreType.DMA((NBUF,)),
                  pltpu.SemaphoreType.DMA((NBUF,))])

    @functools.partial(
        pl.kernel,
        out_type=jax.ShapeDtypeStruct((n, DP), jnp.float32),
        mesh=mesh,
        scratch_types=scratch,
        compiler_params=pltpu.CompilerParams(use_tc_tiling_on_sc=False),
    )
    def k(idx_hbm, table_hbm, out_hbm, *rest):
        idx_v = rest[:NBUF]
        rows_v = rest[NBUF:2 * NBUF]
        gsem, wsem = rest[2 * NBUF], rest[2 * NBUF + 1]
        wid = lax.axis_index("s") * NC + lax.axis_index("c")
        base = wid * b_per_w

        def load_idx(g, b):
            pltpu.sync_copy(idx_hbm.at[pl.ds(base + g * CHUNK, CHUNK)],
                            idx_v[b])

        def start_gather(b):
            pltpu.async_copy(table_hbm.at[idx_v[b]], rows_v[b], gsem.at[b])

        def wait_gather(b):
            pltpu.make_async_copy(table_hbm.at[idx_v[b]], rows_v[b],
                                  gsem.at[b]).wait()

        def start_write(g, b):
            pltpu.async_copy(rows_v[b],
                             out_hbm.at[pl.ds(base + g * CHUNK, CHUNK)],
                             wsem.at[b])

        def wait_write(g, b):
            pltpu.make_async_copy(rows_v[b],
                                  out_hbm.at[pl.ds(base + g * CHUNK, CHUNK)],
                                  wsem.at[b]).wait()

        for b in range(NBUF):
            load_idx(b, b)
            start_gather(b)

        def body(p, carry):
            for b in range(NBUF):
                g = p * NBUF + b
                wait_gather(b)
                start_write(g, b)
                load_idx(g + NBUF, b)
                wait_write(g, b)
                start_gather(b)
            return carry

        lax.fori_loop(0, ngroup - 1, body, 0)

        g0 = (ngroup - 1) * NBUF
        for b in range(NBUF):
            wait_gather(b)
            start_write(g0 + b, b)
        for b in range(NBUF):
            wait_write(g0 + b, b)

    return k(idx, tabp)


def kernel(x, table):
    n = x.shape[0] * x.shape[1]
    idx = x.reshape(-1).astype(jnp.int32)
    tabp = jnp.pad(table, ((0, 0), (0, DP - D)))
    out = _sc_gather(idx, tabp, n)
    return out[:, :D].reshape(x.shape + (table.shape[1],))
